# Initial kernel scaffold; baseline (speedup 1.0000x reference)
import jax
import jax.numpy as jnp
from jax.experimental import pallas as pl


def kernel(c, embX, embY):
    raise NotImplementedError("write your pallas kernel here")


# speedup vs baseline: 3.2073x; 3.2073x over previous
import functools

import jax
import jax.numpy as jnp
from jax import lax
from jax.experimental import pallas as pl
from jax.experimental.pallas import tpu as pltpu
from jax.experimental.pallas import tpu_sc as plsc

GRAPH_NUMBER = 200
WORDS_NUMBER = 100000
OUT_DIM = 64
HALF = OUT_DIM // 2
BATCH = 4096

N_PAIRS = BATCH * GRAPH_NUMBER
NUM_WORKERS = 32
PAIRS_PER_WORKER = N_PAIRS // NUM_WORKERS
CHUNK = 1024
NUM_CHUNKS = PAIRS_PER_WORKER // CHUNK
GROUP = 128
NGROUP = CHUNK // GROUP


def _sc_kernel(c_hbm, embx_hbm, emby_hbm, out_hbm,
               cbuf, xidx, yidx, xbuf, ybuf, sem):
    nc = 2
    wid = lax.axis_index("s") * nc + lax.axis_index("c")
    base = wid * PAIRS_PER_WORKER
    lane = lax.broadcasted_iota(jnp.int32, (16,), 0)
    perm = (lane * 2) % 16
    lo_half = lane < 8

    gdn = lax.GatherDimensionNumbers(
        offset_dims=(), collapsed_slice_dims=(0,), start_index_map=(0,))

    def take16(v, idx):
        return lax.gather(v, idx[:, None], gdn, (1,),
                          mode=lax.GatherScatterMode.PROMISE_IN_BOUNDS)

    def body(t, carry):
        cb = base + t * CHUNK
        pltpu.sync_copy(
            c_hbm.at[pl.ds(pl.multiple_of(cb // 8, 8), 2 * CHUNK // 16)], cbuf)
        for i in range(CHUNK // 16):
            a = cbuf[2 * i]
            b = cbuf[2 * i + 1]
            xv = jnp.where(lo_half, take16(a, perm), take16(b, perm))
            yv = jnp.where(lo_half, take16(a, perm + 1), take16(b, perm + 1))
            xidx[i // 8, pl.ds((i % 8) * 16, 16)] = xv
            yidx[i // 8, pl.ds((i % 8) * 16, 16)] = yv
        copies = []
        for j in range(NGROUP):
            copies.append(pltpu.async_copy(
                embx_hbm.at[xidx.at[j]],
                xbuf.at[pl.ds(j * GROUP, GROUP)], sem))
            copies.append(pltpu.async_copy(
                emby_hbm.at[yidx.at[j]],
                ybuf.at[pl.ds(j * GROUP, GROUP)], sem))
        for cp in copies:
            cp.wait()
        pltpu.sync_copy(xbuf, out_hbm.at[pl.ds(cb, CHUNK), 0])
        pltpu.sync_copy(ybuf, out_hbm.at[pl.ds(cb, CHUNK), 1])
        return carry

    lax.fori_loop(0, NUM_CHUNKS, body, 0)


@jax.jit
def kernel(c, embX, embY):
    c_flat = c.reshape(-1, 16)
    mesh = plsc.VectorSubcoreMesh(core_axis_name="c", subcore_axis_name="s")
    run = functools.partial(
        pl.kernel,
        mesh=mesh,
        out_type=jax.ShapeDtypeStruct((N_PAIRS, 2, HALF), jnp.float32),
        scratch_types=[
            pltpu.VMEM((2 * CHUNK // 16, 16), jnp.int32),
            pltpu.VMEM((NGROUP, GROUP), jnp.int32),
            pltpu.VMEM((NGROUP, GROUP), jnp.int32),
            pltpu.VMEM((CHUNK, HALF), jnp.float32),
            pltpu.VMEM((CHUNK, HALF), jnp.float32),
            pltpu.SemaphoreType.DMA,
        ],
        compiler_params=pltpu.CompilerParams(use_tc_tiling_on_sc=False),
    )(_sc_kernel)
    out = run(c_flat, embX, embY)
    return out.reshape(BATCH, GRAPH_NUMBER, OUT_DIM)


# speedup vs baseline: 3.2479x; 1.0126x over previous
import functools

import jax
import jax.numpy as jnp
from jax import lax
from jax.experimental import pallas as pl
from jax.experimental.pallas import tpu as pltpu
from jax.experimental.pallas import tpu_sc as plsc

GRAPH_NUMBER = 200
WORDS_NUMBER = 100000
OUT_DIM = 64
HALF = OUT_DIM // 2
BATCH = 4096

N_PAIRS = BATCH * GRAPH_NUMBER
NUM_WORKERS = 32
PAIRS_PER_WORKER = N_PAIRS // NUM_WORKERS
CHUNK = 640
NUM_CHUNKS = PAIRS_PER_WORKER // CHUNK
GROUP = 128
NGROUP = CHUNK // GROUP
WRITE_BYTES = 2 * CHUNK * HALF * 4


def _sc_kernel(c_hbm, embx_hbm, emby_hbm, out_hbm,
               cbuf0, cbuf1, xidx0, xidx1, yidx0, yidx1,
               xbuf0, xbuf1, ybuf0, ybuf1, gsem, wsem0, wsem1):
    nc = 2
    wid = lax.axis_index("s") * nc + lax.axis_index("c")
    base = wid * PAIRS_PER_WORKER
    lane = lax.broadcasted_iota(jnp.int32, (16,), 0)
    perm = (lane * 2) % 16
    lo_half = lane < 8
    gdn = lax.GatherDimensionNumbers(
        offset_dims=(), collapsed_slice_dims=(0,), start_index_map=(0,))

    def take16(v, idx):
        return lax.gather(v, idx[:, None], gdn, (1,),
                          mode=lax.GatherScatterMode.PROMISE_IN_BOUNDS)

    def stage(t, cbuf, xidx, yidx):
        cb = base + t * CHUNK
        pltpu.sync_copy(
            c_hbm.at[pl.ds(pl.multiple_of(cb // 8, 8), 2 * CHUNK // 16)],
            cbuf)
        for i in range(CHUNK // 16):
            a = cbuf[2 * i]
            b = cbuf[2 * i + 1]
            xv = jnp.where(lo_half, take16(a, perm), take16(b, perm))
            yv = jnp.where(lo_half, take16(a, perm + 1), take16(b, perm + 1))
            xidx[i // 8, pl.ds((i % 8) * 16, 16)] = xv
            yidx[i // 8, pl.ds((i % 8) * 16, 16)] = yv

    def fire_gathers(xidx, yidx, xbuf, ybuf):
        copies = []
        for j in range(NGROUP):
            copies.append(pltpu.async_copy(
                embx_hbm.at[xidx.at[j]],
                xbuf.at[pl.ds(j * GROUP, GROUP)], gsem))
            copies.append(pltpu.async_copy(
                emby_hbm.at[yidx.at[j]],
                ybuf.at[pl.ds(j * GROUP, GROUP)], gsem))
        return copies

    def fire_writes(t, xbuf, ybuf, wsem):
        cb = base + t * CHUNK
        pltpu.async_copy(xbuf, out_hbm.at[pl.ds(cb, CHUNK), 0], wsem)
        pltpu.async_copy(ybuf, out_hbm.at[pl.ds(cb, CHUNK), 1], wsem)

    def drain_writes(xbuf, ybuf, wsem):
        pltpu.make_async_copy(
            xbuf, out_hbm.at[pl.ds(base, CHUNK), 0], wsem).wait()
        pltpu.make_async_copy(
            ybuf, out_hbm.at[pl.ds(base, CHUNK), 1], wsem).wait()

    stage(0, cbuf0, xidx0, yidx0)

    def body(t2, carry):
        e = 2 * t2
        @pl.when(t2 != 0)
        def _():
            drain_writes(xbuf0, ybuf0, wsem0)
        cps = fire_gathers(xidx0, yidx0, xbuf0, ybuf0)
        stage(e + 1, cbuf1, xidx1, yidx1)
        for cp in cps:
            cp.wait()
        fire_writes(e, xbuf0, ybuf0, wsem0)
        @pl.when(t2 != 0)
        def _():
            drain_writes(xbuf1, ybuf1, wsem1)
        cps = fire_gathers(xidx1, yidx1, xbuf1, ybuf1)

        @pl.when(t2 != NUM_CHUNKS // 2 - 1)
        def _():
            stage(e + 2, cbuf0, xidx0, yidx0)
        for cp in cps:
            cp.wait()
        fire_writes(e + 1, xbuf1, ybuf1, wsem1)
        return carry

    lax.fori_loop(0, NUM_CHUNKS // 2, body, 0)
    drain_writes(xbuf0, ybuf0, wsem0)
    drain_writes(xbuf1, ybuf1, wsem1)


@jax.jit
def kernel(c, embX, embY):
    c_flat = c.reshape(-1, 16)
    mesh = plsc.VectorSubcoreMesh(core_axis_name="c", subcore_axis_name="s")
    run = functools.partial(
        pl.kernel,
        mesh=mesh,
        out_type=jax.ShapeDtypeStruct((N_PAIRS, 2, HALF), jnp.float32),
        scratch_types=[
            pltpu.VMEM((2 * CHUNK // 16, 16), jnp.int32),
            pltpu.VMEM((2 * CHUNK // 16, 16), jnp.int32),
            pltpu.VMEM((NGROUP, GROUP), jnp.int32),
            pltpu.VMEM((NGROUP, GROUP), jnp.int32),
            pltpu.VMEM((NGROUP, GROUP), jnp.int32),
            pltpu.VMEM((NGROUP, GROUP), jnp.int32),
            pltpu.VMEM((CHUNK, HALF), jnp.float32),
            pltpu.VMEM((CHUNK, HALF), jnp.float32),
            pltpu.VMEM((CHUNK, HALF), jnp.float32),
            pltpu.VMEM((CHUNK, HALF), jnp.float32),
            pltpu.SemaphoreType.DMA,
            pltpu.SemaphoreType.DMA,
            pltpu.SemaphoreType.DMA,
        ],
        compiler_params=pltpu.CompilerParams(use_tc_tiling_on_sc=False),
    )(_sc_kernel)
    out = run(c_flat, embX, embY)
    return out.reshape(BATCH, GRAPH_NUMBER, OUT_DIM)
